# baseline (speedup 1.0000x reference)
import functools

import jax
import jax.numpy as jnp
from jax import lax
from jax.experimental import pallas as pl
from jax.experimental.pallas import tpu as pltpu
from jax.experimental.pallas import tpu_sc as plsc


def kernel(labels, token_embedding, tokenized_prompts, ctx):
    B = labels.shape[0]
    C, T = tokenized_prompts.shape
    V, D = token_embedding.shape
    n_ctx = ctx.shape[0]
    n_suf = T - 1 - n_ctx

    info = plsc.get_sparse_core_info()
    NC, NS = info.num_cores, info.num_subcores
    NW = NC * NS
    rows_per_w = B // NW

    W = ((1 + n_suf + 7) // 8) * 8
    toks_tab = jnp.zeros((C, W), jnp.int32)
    toks_tab = toks_tab.at[:, 0].set(tokenized_prompts[:, 0])
    toks_tab = toks_tab.at[:, 1 : 1 + n_suf].set(tokenized_prompts[:, 1 + n_ctx :])

    mesh = plsc.VectorSubcoreMesh(core_axis_name="c", subcore_axis_name="s")

    @functools.partial(
        pl.kernel,
        mesh=mesh,
        out_type=jax.ShapeDtypeStruct((B, T, D), jnp.float32),
        compiler_params=pltpu.CompilerParams(use_tc_tiling_on_sc=False),
        scratch_types=[
            pltpu.VMEM((rows_per_w,), jnp.int32),
            pltpu.VMEM((rows_per_w, W), jnp.int32),
            pltpu.VMEM((n_ctx, D), jnp.float32),
            pltpu.VMEM((3, W, D), jnp.float32),
            pltpu.SemaphoreType.DMA,
            pltpu.SemaphoreType.DMA,
            pltpu.SemaphoreType.DMA,
            pltpu.SemaphoreType.DMA,
            pltpu.SemaphoreType.DMA,
            pltpu.SemaphoreType.DMA,
        ],
    )
    def _prompt_gather(labels_hbm, emb_hbm, toks_hbm, ctx_hbm, out_hbm,
                       labels_v, toks_v, ctx_v, buf,
                       gsem0, gsem1, gsem2, wsem0, wsem1, wsem2):
        wid = lax.axis_index("s") * NC + lax.axis_index("c")
        base = wid * rows_per_w
        gsems = (gsem0, gsem1, gsem2)
        wsems = (wsem0, wsem1, wsem2)

        pltpu.sync_copy(labels_hbm.at[pl.ds(base, rows_per_w)], labels_v)
        pltpu.sync_copy(ctx_hbm, ctx_v)
        pltpu.async_copy(toks_hbm.at[labels_v], toks_v, gsem0).wait()

        def gather_copy(i, s):
            return pltpu.make_async_copy(
                emb_hbm.at[toks_v.at[i]], buf.at[s], gsems[s])

        def write_copies(i, s):
            return (
                pltpu.make_async_copy(
                    buf.at[s, pl.ds(0, 1)],
                    out_hbm.at[base + i, pl.ds(0, 1)], wsems[s]),
                pltpu.make_async_copy(
                    ctx_v, out_hbm.at[base + i, pl.ds(1, n_ctx)], wsems[s]),
                pltpu.make_async_copy(
                    buf.at[s, pl.ds(1, n_suf)],
                    out_hbm.at[base + i, pl.ds(1 + n_ctx, n_suf)], wsems[s]),
            )

        def fire_writes(i, s):
            for c in write_copies(i, s):
                c.start()

        def wait_writes(i, s):
            for c in write_copies(i, s):
                c.wait()

        gather_copy(0, 0).start()
        gather_copy(1, 1).start()
        gather_copy(0, 0).wait()
        fire_writes(0, 0)
        gather_copy(2, 2).start()
        gather_copy(1, 1).wait()
        fire_writes(1, 1)

        def body(k, carry):
            for q in range(3):
                i = 3 * k + 2 + q
                s = (2 + q) % 3
                sn = (s + 1) % 3
                wait_writes(i - 2, sn)
                gather_copy(i + 1, sn).start()
                gather_copy(i, s).wait()
                fire_writes(i, s)
            return carry

        lax.fori_loop(0, (rows_per_w - 5) // 3, body, 0)
        for i, s in ((rows_per_w - 3, 2), (rows_per_w - 2, 0),
                     (rows_per_w - 1, 1)):
            sn = (s + 1) % 3
            wait_writes(i - 2, sn)
            if i + 1 < rows_per_w:
                gather_copy(i + 1, sn).start()
            gather_copy(i, s).wait()
            fire_writes(i, s)
        wait_writes(rows_per_w - 2, 0)
        wait_writes(rows_per_w - 1, 1)

    return _prompt_gather(labels, token_embedding, toks_tab, ctx)


# speedup vs baseline: 1.5532x; 1.5532x over previous
import functools

import jax
import jax.numpy as jnp
from jax import lax
from jax.experimental import pallas as pl
from jax.experimental.pallas import tpu as pltpu
from jax.experimental.pallas import tpu_sc as plsc


def kernel(labels, token_embedding, tokenized_prompts, ctx):
    B = labels.shape[0]
    C, T = tokenized_prompts.shape
    V, D = token_embedding.shape
    n_ctx = ctx.shape[0]

    info = plsc.get_sparse_core_info()
    NC, NS = info.num_cores, info.num_subcores
    NW = NC * NS
    rows_per_w = B // NW

    W = ((T + 7) // 8) * 8
    toks_tab = jnp.concatenate(
        [tokenized_prompts,
         jnp.broadcast_to(tokenized_prompts[:, T - 1 : T], (C, W - T))],
        axis=1)
    toks_tab = toks_tab.at[:, 1 : 1 + n_ctx].set(
        jnp.broadcast_to(tokenized_prompts[:, 0:1], (C, n_ctx)))

    one = (1 - jnp.min(labels) * 0).astype(jnp.float32)
    token_embedding = token_embedding * one
    toks_tab = toks_tab * one.astype(jnp.int32)

    mesh = plsc.VectorSubcoreMesh(core_axis_name="c", subcore_axis_name="s")

    @functools.partial(
        pl.kernel,
        mesh=mesh,
        out_type=jax.ShapeDtypeStruct((B, T, D), jnp.float32),
        compiler_params=pltpu.CompilerParams(use_tc_tiling_on_sc=False),
        scratch_types=[
            pltpu.VMEM((rows_per_w,), jnp.int32),
            pltpu.VMEM((rows_per_w, W), jnp.int32),
            pltpu.VMEM((n_ctx, D), jnp.float32),
            pltpu.VMEM((2, W, D), jnp.float32),
            pltpu.SemaphoreType.DMA,
            pltpu.SemaphoreType.DMA,
            pltpu.SemaphoreType.DMA,
            pltpu.SemaphoreType.DMA,
        ],
    )
    def _prompt_gather(labels_hbm, emb_hbm, toks_hbm, ctx_hbm, out_hbm,
                       labels_v, toks_v, ctx_v, buf,
                       gsem0, gsem1, wsem0, wsem1):
        wid = lax.axis_index("s") * NC + lax.axis_index("c")
        base = wid * rows_per_w
        gsems = (gsem0, gsem1)
        wsems = (wsem0, wsem1)

        pltpu.sync_copy(labels_hbm.at[pl.ds(base, rows_per_w)], labels_v)
        pltpu.sync_copy(ctx_hbm, ctx_v)
        pltpu.async_copy(toks_hbm.at[labels_v], toks_v, gsem0).wait()

        def gather_copy(i, s):
            return pltpu.make_async_copy(
                emb_hbm.at[toks_v.at[i]], buf.at[s], gsems[s])

        def write_copy(i, s):
            return pltpu.make_async_copy(
                buf.at[s, pl.ds(0, T)], out_hbm.at[base + i], wsems[s])

        def patch_ctx(s):
            def prow(r, carry):
                for j in range(D // 16):
                    buf[s, 1 + r, pl.ds(16 * j, 16)] = \
                        ctx_v[r, pl.ds(16 * j, 16)]
                return carry
            lax.fori_loop(0, n_ctx, prow, 0)

        def step(i, s, first, last):
            if not first:
                write_copy(i - 1, 1 - s).wait()
            if not last:
                gather_copy(i + 1, 1 - s).start()
            gather_copy(i, s).wait()
            patch_ctx(s)
            write_copy(i, s).start()

        gather_copy(0, 0).start()
        step(0, 0, True, False)

        def body(k, carry):
            step(2 * k + 1, 1, False, False)
            step(2 * k + 2, 0, False, False)
            return carry

        lax.fori_loop(0, (rows_per_w - 2) // 2, body, 0)
        step(rows_per_w - 1, 1, False, True)
        write_copy(rows_per_w - 1, 1).wait()

    return _prompt_gather(labels, token_embedding, toks_tab, ctx)


# speedup vs baseline: 1.5569x; 1.0024x over previous
import functools

import jax
import jax.numpy as jnp
from jax import lax
from jax.experimental import pallas as pl
from jax.experimental.pallas import tpu as pltpu
from jax.experimental.pallas import tpu_sc as plsc


def kernel(labels, token_embedding, tokenized_prompts, ctx):
    B = labels.shape[0]
    C, T = tokenized_prompts.shape
    V, D = token_embedding.shape
    n_ctx = ctx.shape[0]

    info = plsc.get_sparse_core_info()
    NC, NS = info.num_cores, info.num_subcores
    NW = NC * NS
    rows_per_w = B // NW

    W = ((T + 7) // 8) * 8
    toks_tab = jnp.concatenate(
        [tokenized_prompts,
         jnp.broadcast_to(tokenized_prompts[:, T - 1 : T], (C, W - T))],
        axis=1)
    toks_tab = toks_tab.at[:, 1 : 1 + n_ctx].set(
        jnp.broadcast_to(tokenized_prompts[:, 0:1], (C, n_ctx)))

    one = (1 - jnp.min(labels) * 0).astype(jnp.float32)
    token_embedding = token_embedding * one
    toks_tab = toks_tab * one.astype(jnp.int32)

    mesh = plsc.VectorSubcoreMesh(core_axis_name="c", subcore_axis_name="s")

    @functools.partial(
        pl.kernel,
        mesh=mesh,
        out_type=jax.ShapeDtypeStruct((B * T, D), jnp.float32),
        compiler_params=pltpu.CompilerParams(use_tc_tiling_on_sc=False),
        scratch_types=[
            pltpu.VMEM((rows_per_w,), jnp.int32),
            pltpu.VMEM((rows_per_w, W), jnp.int32),
            pltpu.VMEM((n_ctx, D), jnp.float32),
            pltpu.VMEM((2, W, D), jnp.float32),
            pltpu.SemaphoreType.DMA,
            pltpu.SemaphoreType.DMA,
            pltpu.SemaphoreType.DMA,
            pltpu.SemaphoreType.DMA,
        ],
    )
    def _prompt_gather(labels_hbm, emb_hbm, toks_hbm, ctx_hbm, out_hbm,
                       labels_v, toks_v, ctx_v, buf,
                       gsem0, gsem1, wsem0, wsem1):
        wid = lax.axis_index("s") * NC + lax.axis_index("c")
        base = wid * rows_per_w
        gsems = (gsem0, gsem1)
        wsems = (wsem0, wsem1)

        pltpu.sync_copy(labels_hbm.at[pl.ds(base, rows_per_w)], labels_v)
        pltpu.sync_copy(ctx_hbm, ctx_v)
        pltpu.async_copy(toks_hbm.at[labels_v], toks_v, gsem0).wait()

        def gather_copy(i, s):
            return pltpu.make_async_copy(
                emb_hbm.at[toks_v.at[i]], buf.at[s], gsems[s])

        def write_copy(i, s):
            return pltpu.make_async_copy(
                buf.at[s, pl.ds(0, T)],
                out_hbm.at[pl.ds((base + i) * T, T)], wsems[s])

        def patch_ctx(s):
            def prow(r, carry):
                for j in range(D // 16):
                    buf[s, 1 + r, pl.ds(16 * j, 16)] = \
                        ctx_v[r, pl.ds(16 * j, 16)]
                return carry
            lax.fori_loop(0, n_ctx, prow, 0)

        def step(i, s, first, last):
            if not first:
                write_copy(i - 1, 1 - s).wait()
            if not last:
                gather_copy(i + 1, 1 - s).start()
            gather_copy(i, s).wait()
            patch_ctx(s)
            write_copy(i, s).start()

        gather_copy(0, 0).start()
        step(0, 0, True, False)

        def body(k, carry):
            step(2 * k + 1, 1, False, False)
            step(2 * k + 2, 0, False, False)
            return carry

        lax.fori_loop(0, (rows_per_w - 2) // 2, body, 0)
        step(rows_per_w - 1, 1, False, True)
        write_copy(rows_per_w - 1, 1).wait()

    return _prompt_gather(labels, token_embedding, toks_tab, ctx).reshape(
        B, T, D)


# speedup vs baseline: 1.5576x; 1.0005x over previous
import functools

import jax
import jax.numpy as jnp
from jax import lax
from jax.experimental import pallas as pl
from jax.experimental.pallas import tpu as pltpu
from jax.experimental.pallas import tpu_sc as plsc


def kernel(labels, token_embedding, tokenized_prompts, ctx):
    B = labels.shape[0]
    C, T = tokenized_prompts.shape
    V, D = token_embedding.shape
    n_ctx = ctx.shape[0]

    info = plsc.get_sparse_core_info()
    NC, NS = info.num_cores, info.num_subcores
    NW = NC * NS
    rows_per_w = B // NW

    W = ((T + 7) // 8) * 8
    toks_tab = jnp.concatenate(
        [tokenized_prompts,
         jnp.broadcast_to(tokenized_prompts[:, T - 1 : T], (C, W - T))],
        axis=1)
    toks_tab = toks_tab.at[:, 1 : 1 + n_ctx].set(
        jnp.broadcast_to(tokenized_prompts[:, 0:1], (C, n_ctx)))

    mesh = plsc.VectorSubcoreMesh(core_axis_name="c", subcore_axis_name="s")

    @functools.partial(
        pl.kernel,
        mesh=mesh,
        out_type=jax.ShapeDtypeStruct((B, T, D), jnp.float32),
        compiler_params=pltpu.CompilerParams(use_tc_tiling_on_sc=False),
        scratch_types=[
            pltpu.VMEM((rows_per_w,), jnp.int32),
            pltpu.VMEM((rows_per_w, W), jnp.int32),
            pltpu.VMEM((n_ctx, D), jnp.float32),
            pltpu.VMEM((2, W, D), jnp.float32),
            pltpu.SemaphoreType.DMA,
            pltpu.SemaphoreType.DMA,
            pltpu.SemaphoreType.DMA,
            pltpu.SemaphoreType.DMA,
        ],
    )
    def _prompt_gather(labels_hbm, emb_hbm, toks_hbm, ctx_hbm, out_hbm,
                       labels_v, toks_v, ctx_v, buf,
                       gsem0, gsem1, wsem0, wsem1):
        wid = lax.axis_index("s") * NC + lax.axis_index("c")
        base = wid * rows_per_w
        gsems = (gsem0, gsem1)
        wsems = (wsem0, wsem1)

        pltpu.sync_copy(labels_hbm.at[pl.ds(base, rows_per_w)], labels_v)
        pltpu.sync_copy(ctx_hbm, ctx_v)
        pltpu.async_copy(toks_hbm.at[labels_v], toks_v, gsem0).wait()

        def gather_copy(i, s):
            return pltpu.make_async_copy(
                emb_hbm.at[toks_v.at[i]], buf.at[s], gsems[s])

        def write_copy(i, s):
            return pltpu.make_async_copy(
                buf.at[s, pl.ds(0, T)], out_hbm.at[base + i], wsems[s])

        def patch_ctx(s):
            def prow(r, carry):
                for j in range(D // 16):
                    buf[s, 1 + r, pl.ds(16 * j, 16)] = \
                        ctx_v[r, pl.ds(16 * j, 16)]
                return carry
            lax.fori_loop(0, n_ctx, prow, 0)

        def step(i, s, first, last):
            if not first:
                write_copy(i - 1, 1 - s).wait()
            if not last:
                gather_copy(i + 1, 1 - s).start()
            gather_copy(i, s).wait()
            patch_ctx(s)
            write_copy(i, s).start()

        gather_copy(0, 0).start()
        step(0, 0, True, False)

        def body(k, carry):
            step(2 * k + 1, 1, False, False)
            step(2 * k + 2, 0, False, False)
            return carry

        lax.fori_loop(0, (rows_per_w - 2) // 2, body, 0)
        step(rows_per_w - 1, 1, False, True)
        write_copy(rows_per_w - 1, 1).wait()

    return _prompt_gather(labels, token_embedding, toks_tab, ctx)
